# baseline (speedup 1.0000x reference)
import jax
import jax.numpy as jnp
from jax import lax
from jax.experimental import pallas as pl
from jax.experimental.pallas import tpu as pltpu
from jax.experimental.pallas import tpu_sc as plsc

NC = 2
NS = 16
NW = NC * NS
G = 128
D = 128
B = 400000
R_ALL = B // G
R = 2720
NBUF = 7
STEPS = 91


def _gather_body(table_hbm, idx_hbm, out_hbm, idx_r, rows_r, *sems):
    isems = sems[:NBUF]
    gsems = sems[NBUF:2 * NBUF]
    wsems = sems[2 * NBUF:]
    wid = lax.axis_index("s") * NC + lax.axis_index("c")

    def valid(i):
        return wid + i * NW < R

    def istart(i, b):
        pltpu.async_copy(idx_hbm.at[wid + i * NW], idx_r.at[b], isems[b])

    def iwait(b):
        pltpu.make_async_copy(idx_hbm.at[0], idx_r.at[b], isems[b]).wait()

    def gstart(b):
        pltpu.async_copy(table_hbm.at[idx_r.at[b]], rows_r.at[b], gsems[b])

    def gwait(b):
        pltpu.make_async_copy(
            table_hbm.at[pl.ds(0, G)], rows_r.at[b], gsems[b]).wait()

    def wstart(i, b):
        pltpu.async_copy(
            rows_r.at[b], out_hbm.at[pl.ds((wid + i * NW) * G, G)], wsems[b])

    def wwait(b):
        pltpu.make_async_copy(
            rows_r.at[b], out_hbm.at[pl.ds(0, G)], wsems[b]).wait()

    for c in range(4):
        @pl.when(valid(c))
        def _():
            istart(c, c)

    for c in range(2):
        @pl.when(valid(c))
        def _():
            iwait(c)
            gstart(c)

    def step(g, carry):
        for b in range(NBUF):
            i = lambda off=0: g * NBUF + b + off
            b2 = (b + 2) % NBUF
            b4 = (b + 4) % NBUF

            @pl.when(valid(i(-5)) & (i() >= 5))
            def _():
                wwait(b2)

            @pl.when(valid(i(4)))
            def _():
                istart(i(4), b4)

            @pl.when(valid(i(2)))
            def _():
                iwait(b2)
                gstart(b2)

            @pl.when(valid(i()))
            def _():
                gwait(b)
                wstart(i(), b)

        return carry

    lax.fori_loop(0, STEPS // NBUF, step, 0)

    for c in range(STEPS - 5, STEPS):
        @pl.when(valid(c))
        def _():
            wwait(c % NBUF)


def kernel(x, upsample):
    idx = upsample[:, 0].astype(jnp.int32)
    idx2 = idx[:R * G].reshape(R, G)
    table = jnp.concatenate([x, jnp.zeros((1, D), x.dtype)], axis=0)
    f = pl.kernel(
        _gather_body,
        out_type=jax.ShapeDtypeStruct((B, D), jnp.float32),
        mesh=plsc.VectorSubcoreMesh(core_axis_name="c", subcore_axis_name="s"),
        scratch_types=(
            [pltpu.VMEM((NBUF, G), jnp.int32),
             pltpu.VMEM((NBUF, G, D), jnp.float32)]
            + [pltpu.SemaphoreType.DMA] * (3 * NBUF)
        ),
    )
    out_sc = f(table, idx2)
    tc_rows = jnp.take(table, idx[R * G:], axis=0)
    return lax.dynamic_update_slice(out_sc, tc_rows, (R * G, 0))


# speedup vs baseline: 1.1937x; 1.1937x over previous
import jax
import jax.numpy as jnp
from jax import lax
from jax.experimental import pallas as pl
from jax.experimental.pallas import tpu as pltpu
from jax.experimental.pallas import tpu_sc as plsc

NC = 2
NS = 16
NW = NC * NS
G = 128
D = 128
B = 400000
R = B // G
NIT = (R + NW - 1) // NW
NBUF = 7
STEPS = 98


def _gather_body(table_hbm, idx_hbm, out_hbm, idx_r, rows_r, *sems):
    isems = sems[:NBUF]
    gsems = sems[NBUF:2 * NBUF]
    wsems = sems[2 * NBUF:]
    wid = lax.axis_index("s") * NC + lax.axis_index("c")

    def valid(i):
        return wid + i * NW < R

    def istart(i, b):
        pltpu.async_copy(idx_hbm.at[wid + i * NW], idx_r.at[b], isems[b])

    def iwait(b):
        pltpu.make_async_copy(idx_hbm.at[0], idx_r.at[b], isems[b]).wait()

    def gstart(b):
        pltpu.async_copy(table_hbm.at[idx_r.at[b]], rows_r.at[b], gsems[b])

    def gwait(b):
        pltpu.make_async_copy(
            table_hbm.at[pl.ds(0, G)], rows_r.at[b], gsems[b]).wait()

    def wstart(i, b):
        pltpu.async_copy(
            rows_r.at[b], out_hbm.at[pl.ds((wid + i * NW) * G, G)], wsems[b])

    def wwait(b):
        pltpu.make_async_copy(
            rows_r.at[b], out_hbm.at[pl.ds(0, G)], wsems[b]).wait()

    for c in range(4):
        @pl.when(valid(c))
        def _():
            istart(c, c)

    for c in range(2):
        @pl.when(valid(c))
        def _():
            iwait(c)
            gstart(c)

    def step(g, carry):
        for b in range(NBUF):
            i = lambda off=0: g * NBUF + b + off
            b2 = (b + 2) % NBUF
            b4 = (b + 4) % NBUF

            @pl.when(valid(i(-5)) & (i() >= 5))
            def _():
                wwait(b2)

            @pl.when(valid(i(4)))
            def _():
                istart(i(4), b4)

            @pl.when(valid(i(2)))
            def _():
                iwait(b2)
                gstart(b2)

            @pl.when(valid(i()))
            def _():
                gwait(b)
                wstart(i(), b)

        return carry

    lax.fori_loop(0, STEPS // NBUF, step, 0)

    for c in range(STEPS - 5, STEPS):
        @pl.when(valid(c))
        def _():
            wwait(c % NBUF)


def kernel(x, upsample):
    idx = upsample[:, 0].astype(jnp.int32).reshape(R, G)
    table = jnp.concatenate([x, jnp.zeros((1, D), x.dtype)], axis=0)
    f = pl.kernel(
        _gather_body,
        out_type=jax.ShapeDtypeStruct((B, D), jnp.float32),
        mesh=plsc.VectorSubcoreMesh(core_axis_name="c", subcore_axis_name="s"),
        scratch_types=(
            [pltpu.VMEM((NBUF, G), jnp.int32),
             pltpu.VMEM((NBUF, G, D), jnp.float32)]
            + [pltpu.SemaphoreType.DMA] * (3 * NBUF)
        ),
    )
    return f(table, idx)
